# baseline (speedup 1.0000x reference)
import functools

import jax
import jax.numpy as jnp
from jax import lax
from jax.experimental import pallas as pl
from jax.experimental.pallas import tpu as pltpu
from jax.experimental.pallas import tpu_sc as plsc

_NC = 2
_NS = 16
_CH = 32
_NBUF = 3
_TC_CHUNK = 2048


def _tc_body(t_ref, o_ref):
    o_ref[0] = t_ref[...]


def _tc_copy(table, B, S, D):
    return pl.pallas_call(
        _tc_body,
        grid=(S // _TC_CHUNK, B),
        in_specs=[pl.BlockSpec((_TC_CHUNK, D), lambda i, b: (i, 0))],
        out_specs=pl.BlockSpec((1, _TC_CHUNK, D), lambda i, b: (b, i, 0)),
        out_shape=jax.ShapeDtypeStruct((B, S, D), table.dtype),
    )(table)


def _sc_copy(table, B, S, D):
    NW = _NC * _NS
    rows = S // NW
    nchunk = rows // _CH
    mesh = plsc.VectorSubcoreMesh(core_axis_name="c", subcore_axis_name="s")

    @functools.partial(
        pl.kernel,
        out_type=jax.ShapeDtypeStruct((B, S, D), table.dtype),
        mesh=mesh,
        scratch_types=(
            [pltpu.VMEM((_CH, D), table.dtype) for _ in range(_NBUF)]
            + [pltpu.SemaphoreType.DMA, pltpu.SemaphoreType.DMA]
        ),
    )
    def body(table_hbm, out_hbm, *rest):
        bufs, (in_sem, out_sem) = list(rest[:_NBUF]), rest[_NBUF:]
        wid = lax.axis_index("s") * _NC + lax.axis_index("c")
        base = wid * rows

        def start_in(i):
            return pltpu.async_copy(
                table_hbm.at[pl.ds(base + i * _CH, _CH)],
                bufs[i % _NBUF], in_sem)

        in_copies = [None] * nchunk
        out_copies = [None] * nchunk
        drained = [False] * nchunk
        in_copies[0] = start_in(0)
        for i in range(nchunk):
            in_copies[i].wait()
            out_copies[i] = [
                pltpu.async_copy(
                    bufs[i % _NBUF],
                    out_hbm.at[b].at[pl.ds(base + i * _CH, _CH)],
                    out_sem)
                for b in range(B)
            ]
            if i + 1 < nchunk:
                prev_user = i + 1 - _NBUF
                if prev_user >= 0:
                    for c in out_copies[prev_user]:
                        c.wait()
                    drained[prev_user] = True
                in_copies[i + 1] = start_in(i + 1)
        for i in range(nchunk):
            if not drained[i]:
                for c in out_copies[i]:
                    c.wait()

    return body(table)


def kernel(x, table):
    B, S, D = x.shape
    return _tc_copy(table, B, S, D)


# speedup vs baseline: 1.1007x; 1.1007x over previous
import functools

import jax
import jax.numpy as jnp
from jax import lax
from jax.experimental import pallas as pl
from jax.experimental.pallas import tpu as pltpu
from jax.experimental.pallas import tpu_sc as plsc

_NC = 2
_NS = 16
_CH = 32
_NBUF = 3
_TC_CHUNK = 2048


def _tc_body(t_ref, o_ref):
    o_ref[...] = jnp.broadcast_to(t_ref[...][None], o_ref.shape)


def _tc_copy(table, B, S, D):
    return pl.pallas_call(
        _tc_body,
        grid=(S // _TC_CHUNK,),
        in_specs=[pl.BlockSpec((_TC_CHUNK, D), lambda i: (i, 0))],
        out_specs=pl.BlockSpec((B, _TC_CHUNK, D), lambda i: (0, i, 0)),
        out_shape=jax.ShapeDtypeStruct((B, S, D), table.dtype),
    )(table)


_MCH = 512
_MNBUF = 4


def _tc_manual_copy(table, B, S, D):
    nchunk = S // _MCH

    def body(t_hbm, o_hbm, bufs, in_sem, out_sem):
        def start_in(i):
            c = pltpu.make_async_copy(
                t_hbm.at[pl.ds(i * _MCH, _MCH)],
                bufs.at[i % _MNBUF],
                in_sem.at[i % _MNBUF])
            c.start()
            return c

        def make_outs(i):
            return [
                pltpu.make_async_copy(
                    bufs.at[i % _MNBUF],
                    o_hbm.at[b, pl.ds(i * _MCH, _MCH)],
                    out_sem.at[i % _MNBUF])
                for b in range(B)
            ]

        P = 2
        in_copies = [None] * nchunk
        out_copies = [None] * nchunk
        drained = [False] * nchunk
        for i in range(min(P, nchunk)):
            in_copies[i] = start_in(i)
        for i in range(nchunk):
            in_copies[i].wait()
            out_copies[i] = make_outs(i)
            for c in out_copies[i]:
                c.start()
            nxt = i + P
            if nxt < nchunk:
                prev_user = nxt - _MNBUF
                if prev_user >= 0:
                    for c in out_copies[prev_user]:
                        c.wait()
                    drained[prev_user] = True
                in_copies[nxt] = start_in(nxt)
        for i in range(nchunk):
            if not drained[i]:
                for c in out_copies[i]:
                    c.wait()

    return pl.pallas_call(
        body,
        in_specs=[pl.BlockSpec(memory_space=pl.ANY)],
        out_specs=pl.BlockSpec(memory_space=pl.ANY),
        out_shape=jax.ShapeDtypeStruct((B, S, D), table.dtype),
        scratch_shapes=[
            pltpu.VMEM((_MNBUF, _MCH, D), table.dtype),
            pltpu.SemaphoreType.DMA((_MNBUF,)),
            pltpu.SemaphoreType.DMA((_MNBUF,)),
        ],
    )(table)


def _sc_copy(table, B, S, D):
    NW = _NC * _NS
    rows = S // NW
    nchunk = rows // _CH
    mesh = plsc.VectorSubcoreMesh(core_axis_name="c", subcore_axis_name="s")

    @functools.partial(
        pl.kernel,
        out_type=jax.ShapeDtypeStruct((B, S, D), table.dtype),
        mesh=mesh,
        scratch_types=(
            [pltpu.VMEM((_CH, D), table.dtype) for _ in range(_NBUF)]
            + [pltpu.SemaphoreType.DMA, pltpu.SemaphoreType.DMA]
        ),
    )
    def body(table_hbm, out_hbm, *rest):
        bufs, (in_sem, out_sem) = list(rest[:_NBUF]), rest[_NBUF:]
        wid = lax.axis_index("s") * _NC + lax.axis_index("c")
        base = wid * rows

        def start_in(i):
            return pltpu.async_copy(
                table_hbm.at[pl.ds(base + i * _CH, _CH)],
                bufs[i % _NBUF], in_sem)

        in_copies = [None] * nchunk
        out_copies = [None] * nchunk
        drained = [False] * nchunk
        in_copies[0] = start_in(0)
        for i in range(nchunk):
            in_copies[i].wait()
            out_copies[i] = [
                pltpu.async_copy(
                    bufs[i % _NBUF],
                    out_hbm.at[b].at[pl.ds(base + i * _CH, _CH)],
                    out_sem)
                for b in range(B)
            ]
            if i + 1 < nchunk:
                prev_user = i + 1 - _NBUF
                if prev_user >= 0:
                    for c in out_copies[prev_user]:
                        c.wait()
                    drained[prev_user] = True
                in_copies[i + 1] = start_in(i + 1)
        for i in range(nchunk):
            if not drained[i]:
                for c in out_copies[i]:
                    c.wait()

    return body(table)


def kernel(x, table):
    B, S, D = x.shape
    return _tc_manual_copy(table, B, S, D)


# speedup vs baseline: 1.1336x; 1.0298x over previous
import functools

import jax
import jax.numpy as jnp
from jax import lax
from jax.experimental import pallas as pl
from jax.experimental.pallas import tpu as pltpu
from jax.experimental.pallas import tpu_sc as plsc

_NC = 2
_NS = 16
_CH = 32
_NBUF = 3
_TC_CHUNK = 2048


def _tc_body(t_ref, o_ref):
    o_ref[...] = jnp.broadcast_to(t_ref[...][None], o_ref.shape)


def _tc_copy(table, B, S, D):
    return pl.pallas_call(
        _tc_body,
        grid=(S // _TC_CHUNK,),
        in_specs=[pl.BlockSpec((_TC_CHUNK, D), lambda i: (i, 0))],
        out_specs=pl.BlockSpec((B, _TC_CHUNK, D), lambda i: (0, i, 0)),
        out_shape=jax.ShapeDtypeStruct((B, S, D), table.dtype),
    )(table)


_MCH = 1024
_MNBUF = 4


def _tc_manual_copy(table, B, S, D):
    nchunk = S // _MCH

    def body(t_hbm, o_hbm, bufs, in_sem, out_sem):
        def start_in(i):
            c = pltpu.make_async_copy(
                t_hbm.at[pl.ds(i * _MCH, _MCH)],
                bufs.at[i % _MNBUF],
                in_sem.at[i % _MNBUF])
            c.start()
            return c

        def make_outs(i):
            return [
                pltpu.make_async_copy(
                    bufs.at[i % _MNBUF],
                    o_hbm.at[b, pl.ds(i * _MCH, _MCH)],
                    out_sem.at[i % _MNBUF])
                for b in range(B)
            ]

        P = 2
        in_copies = [None] * nchunk
        out_copies = [None] * nchunk
        drained = [False] * nchunk
        for i in range(min(P, nchunk)):
            in_copies[i] = start_in(i)
        for i in range(nchunk):
            in_copies[i].wait()
            out_copies[i] = make_outs(i)
            for c in out_copies[i]:
                c.start()
            nxt = i + P
            if nxt < nchunk:
                prev_user = nxt - _MNBUF
                if prev_user >= 0:
                    for c in out_copies[prev_user]:
                        c.wait()
                    drained[prev_user] = True
                in_copies[nxt] = start_in(nxt)
        for i in range(nchunk):
            if not drained[i]:
                for c in out_copies[i]:
                    c.wait()

    return pl.pallas_call(
        body,
        in_specs=[pl.BlockSpec(memory_space=pl.ANY)],
        out_specs=pl.BlockSpec(memory_space=pl.ANY),
        out_shape=jax.ShapeDtypeStruct((B, S, D), table.dtype),
        scratch_shapes=[
            pltpu.VMEM((_MNBUF, _MCH, D), table.dtype),
            pltpu.SemaphoreType.DMA((_MNBUF,)),
            pltpu.SemaphoreType.DMA((_MNBUF,)),
        ],
    )(table)


def _sc_copy(table, B, S, D):
    NW = _NC * _NS
    rows = S // NW
    nchunk = rows // _CH
    mesh = plsc.VectorSubcoreMesh(core_axis_name="c", subcore_axis_name="s")

    @functools.partial(
        pl.kernel,
        out_type=jax.ShapeDtypeStruct((B, S, D), table.dtype),
        mesh=mesh,
        scratch_types=(
            [pltpu.VMEM((_CH, D), table.dtype) for _ in range(_NBUF)]
            + [pltpu.SemaphoreType.DMA, pltpu.SemaphoreType.DMA]
        ),
    )
    def body(table_hbm, out_hbm, *rest):
        bufs, (in_sem, out_sem) = list(rest[:_NBUF]), rest[_NBUF:]
        wid = lax.axis_index("s") * _NC + lax.axis_index("c")
        base = wid * rows

        def start_in(i):
            return pltpu.async_copy(
                table_hbm.at[pl.ds(base + i * _CH, _CH)],
                bufs[i % _NBUF], in_sem)

        in_copies = [None] * nchunk
        out_copies = [None] * nchunk
        drained = [False] * nchunk
        in_copies[0] = start_in(0)
        for i in range(nchunk):
            in_copies[i].wait()
            out_copies[i] = [
                pltpu.async_copy(
                    bufs[i % _NBUF],
                    out_hbm.at[b].at[pl.ds(base + i * _CH, _CH)],
                    out_sem)
                for b in range(B)
            ]
            if i + 1 < nchunk:
                prev_user = i + 1 - _NBUF
                if prev_user >= 0:
                    for c in out_copies[prev_user]:
                        c.wait()
                    drained[prev_user] = True
                in_copies[i + 1] = start_in(i + 1)
        for i in range(nchunk):
            if not drained[i]:
                for c in out_copies[i]:
                    c.wait()

    return body(table)


def kernel(x, table):
    B, S, D = x.shape
    return _tc_manual_copy(table, B, S, D)


# speedup vs baseline: 1.1638x; 1.0267x over previous
import functools

import jax
import jax.numpy as jnp
from jax import lax
from jax.experimental import pallas as pl
from jax.experimental.pallas import tpu as pltpu
from jax.experimental.pallas import tpu_sc as plsc

_NC = 2
_NS = 16
_CH = 32
_NBUF = 3
_TC_CHUNK = 2048


def _tc_body(t_ref, o_ref):
    o_ref[...] = jnp.broadcast_to(t_ref[...][None], o_ref.shape)


def _tc_copy(table, B, S, D):
    return pl.pallas_call(
        _tc_body,
        grid=(S // _TC_CHUNK,),
        in_specs=[pl.BlockSpec((_TC_CHUNK, D), lambda i: (i, 0))],
        out_specs=pl.BlockSpec((B, _TC_CHUNK, D), lambda i: (0, i, 0)),
        out_shape=jax.ShapeDtypeStruct((B, S, D), table.dtype),
    )(table)


_MCH = 2048
_MNBUF = 4


def _tc_manual_copy(table, B, S, D):
    nchunk = S // _MCH

    def body(t_hbm, o_hbm, bufs, in_sem, out_sem):
        def start_in(i):
            c = pltpu.make_async_copy(
                t_hbm.at[pl.ds(i * _MCH, _MCH)],
                bufs.at[i % _MNBUF],
                in_sem.at[i % _MNBUF])
            c.start()
            return c

        def make_outs(i):
            return [
                pltpu.make_async_copy(
                    bufs.at[i % _MNBUF],
                    o_hbm.at[b, pl.ds(i * _MCH, _MCH)],
                    out_sem.at[i % _MNBUF])
                for b in range(B)
            ]

        P = 2
        in_copies = [None] * nchunk
        out_copies = [None] * nchunk
        drained = [False] * nchunk
        for i in range(min(P, nchunk)):
            in_copies[i] = start_in(i)
        for i in range(nchunk):
            in_copies[i].wait()
            out_copies[i] = make_outs(i)
            for c in out_copies[i]:
                c.start()
            nxt = i + P
            if nxt < nchunk:
                prev_user = nxt - _MNBUF
                if prev_user >= 0:
                    for c in out_copies[prev_user]:
                        c.wait()
                    drained[prev_user] = True
                in_copies[nxt] = start_in(nxt)
        for i in range(nchunk):
            if not drained[i]:
                for c in out_copies[i]:
                    c.wait()

    return pl.pallas_call(
        body,
        in_specs=[pl.BlockSpec(memory_space=pl.ANY)],
        out_specs=pl.BlockSpec(memory_space=pl.ANY),
        out_shape=jax.ShapeDtypeStruct((B, S, D), table.dtype),
        scratch_shapes=[
            pltpu.VMEM((_MNBUF, _MCH, D), table.dtype),
            pltpu.SemaphoreType.DMA((_MNBUF,)),
            pltpu.SemaphoreType.DMA((_MNBUF,)),
        ],
    )(table)


def _sc_copy(table, B, S, D):
    NW = _NC * _NS
    rows = S // NW
    nchunk = rows // _CH
    mesh = plsc.VectorSubcoreMesh(core_axis_name="c", subcore_axis_name="s")

    @functools.partial(
        pl.kernel,
        out_type=jax.ShapeDtypeStruct((B, S, D), table.dtype),
        mesh=mesh,
        scratch_types=(
            [pltpu.VMEM((_CH, D), table.dtype) for _ in range(_NBUF)]
            + [pltpu.SemaphoreType.DMA, pltpu.SemaphoreType.DMA]
        ),
    )
    def body(table_hbm, out_hbm, *rest):
        bufs, (in_sem, out_sem) = list(rest[:_NBUF]), rest[_NBUF:]
        wid = lax.axis_index("s") * _NC + lax.axis_index("c")
        base = wid * rows

        def start_in(i):
            return pltpu.async_copy(
                table_hbm.at[pl.ds(base + i * _CH, _CH)],
                bufs[i % _NBUF], in_sem)

        in_copies = [None] * nchunk
        out_copies = [None] * nchunk
        drained = [False] * nchunk
        in_copies[0] = start_in(0)
        for i in range(nchunk):
            in_copies[i].wait()
            out_copies[i] = [
                pltpu.async_copy(
                    bufs[i % _NBUF],
                    out_hbm.at[b].at[pl.ds(base + i * _CH, _CH)],
                    out_sem)
                for b in range(B)
            ]
            if i + 1 < nchunk:
                prev_user = i + 1 - _NBUF
                if prev_user >= 0:
                    for c in out_copies[prev_user]:
                        c.wait()
                    drained[prev_user] = True
                in_copies[i + 1] = start_in(i + 1)
        for i in range(nchunk):
            if not drained[i]:
                for c in out_copies[i]:
                    c.wait()

    return body(table)


def kernel(x, table):
    B, S, D = x.shape
    return _tc_manual_copy(table, B, S, D)
